# Initial kernel scaffold; baseline (speedup 1.0000x reference)
import jax
import jax.numpy as jnp
from jax.experimental import pallas as pl


def kernel(x, Wr, br, W1, b1, W2, b2):
    raise NotImplementedError("write your pallas kernel here")


# speedup vs baseline: 1.2549x; 1.2549x over previous
import functools

import jax
import jax.numpy as jnp
from jax import lax
from jax.experimental import pallas as pl
from jax.experimental.pallas import tpu as pltpu
from jax.experimental.pallas import tpu_sc as plsc

TM = 256
FT = 2048
LANES = 128


def _router_body(x_ref, wr_ref, br_ref, top1_ref, prob_ref, aux_ref):
    n, e_pad = top1_ref.shape
    x = x_ref[...]
    logits = jnp.dot(x, wr_ref[...], preferred_element_type=jnp.float32,
                     precision=lax.Precision.DEFAULT) + br_ref[...]
    m = jnp.max(logits, axis=1, keepdims=True)
    p = jnp.exp(logits - m)
    s = jnp.sum(p, axis=1, keepdims=True)
    lane = lax.broadcasted_iota(jnp.int32, (n, e_pad), 1)
    top1 = jnp.min(jnp.where(logits >= m, lane, e_pad), axis=1, keepdims=True)
    top1_ref[...] = jnp.broadcast_to(top1, (n, e_pad))
    prob_ref[...] = jnp.broadcast_to(1.0 / s, (n, e_pad))
    probs = p / s
    importance = jnp.sum(probs, axis=0, keepdims=True) * (1.0 / n)
    load = jnp.sum(jnp.where(lane == top1, 1.0, 0.0), axis=0, keepdims=True) * (1.0 / n)
    e_real = aux_ref.shape[0]
    aux = e_real * jnp.sum(importance * load)
    aux_ref[...] = jnp.broadcast_to(aux, aux_ref.shape)


def _run_router(x2d, wr, br):
    n, d = x2d.shape
    e = wr.shape[1]
    wr_pad = jnp.zeros((d, LANES), jnp.float32).at[:, :e].set(wr)
    br_pad = jnp.full((1, LANES), -1e30, jnp.float32).at[0, :e].set(br)
    return pl.pallas_call(
        _router_body,
        out_shape=(
            jax.ShapeDtypeStruct((n, LANES), jnp.int32),
            jax.ShapeDtypeStruct((n, LANES), jnp.float32),
            jax.ShapeDtypeStruct((e, LANES), jnp.float32),
        ),
    )(x2d, wr_pad, br_pad)


def _sc_gather(table, idx, chunk):
    rows, d = table.shape
    b = idx.shape[0]
    info = plsc.get_sparse_core_info()
    nw = info.num_cores * info.num_subcores
    b_per_w = b // nw
    n_chunks = b_per_w // chunk
    assert b_per_w * nw == b and n_chunks * chunk == b_per_w
    mesh = plsc.VectorSubcoreMesh(core_axis_name="c", subcore_axis_name="s")

    @functools.partial(
        pl.kernel, mesh=mesh,
        out_type=jax.ShapeDtypeStruct((b, d), jnp.float32),
        scratch_types=[
            pltpu.VMEM((chunk,), jnp.int32),
            pltpu.VMEM((chunk, d), jnp.float32),
            pltpu.SemaphoreType.DMA,
        ],
    )
    def k(table_hbm, idx_hbm, out_hbm, idx_v, rows_v, sem):
        wid = lax.axis_index("s") * info.num_cores + lax.axis_index("c")
        base = wid * b_per_w
        for i in range(n_chunks):
            off = base + i * chunk
            pltpu.sync_copy(idx_hbm.at[pl.ds(off, chunk)], idx_v)
            pltpu.async_copy(table_hbm.at[idx_v], rows_v, sem).wait()
            pltpu.sync_copy(rows_v, out_hbm.at[pl.ds(off, chunk)])

    return k(table, idx)


def _ffn_body(be_ref, x_ref, w1_ref, b1_ref, w2_ref, b2_ref, gate_ref,
              out_ref, acc_ref, *, kt_total):
    kt = pl.program_id(1)
    x = x_ref[...].astype(jnp.bfloat16)
    w1 = w1_ref[0].astype(jnp.bfloat16)
    h = jnp.dot(x, w1, preferred_element_type=jnp.float32) + b1_ref[0]
    h = h * (1.0 / (1.0 + jnp.exp(-h)))
    w2 = w2_ref[0].astype(jnp.bfloat16)
    part = jnp.dot(h.astype(jnp.bfloat16), w2, preferred_element_type=jnp.float32)

    @pl.when(kt == 0)
    def _():
        acc_ref[...] = part

    @pl.when(kt > 0)
    def _():
        acc_ref[...] = acc_ref[...] + part

    @pl.when(kt == kt_total - 1)
    def _():
        g = gate_ref[:, 0:1]
        out_ref[...] = (acc_ref[...] + b2_ref[0]) * g


def _run_ffn(x_sorted, w1, b1, w2, b2, gate_mat, block_expert, num_blocks):
    n_pad, d = x_sorted.shape
    e, _, f = w1.shape
    kt_total = f // FT
    b1r = b1.reshape(e, 1, f)
    b2r = b2.reshape(e, 1, d)
    grid_spec = pltpu.PrefetchScalarGridSpec(
        num_scalar_prefetch=1,
        grid=(num_blocks, kt_total),
        in_specs=[
            pl.BlockSpec((TM, d), lambda b, kt, be: (b, 0)),
            pl.BlockSpec((1, d, FT), lambda b, kt, be: (be[b], 0, kt)),
            pl.BlockSpec((1, 1, FT), lambda b, kt, be: (be[b], 0, kt)),
            pl.BlockSpec((1, FT, d), lambda b, kt, be: (be[b], kt, 0)),
            pl.BlockSpec((1, 1, d), lambda b, kt, be: (be[b], 0, 0)),
            pl.BlockSpec((TM, LANES), lambda b, kt, be: (b, 0)),
        ],
        out_specs=pl.BlockSpec((TM, d), lambda b, kt, be: (b, 0)),
        scratch_shapes=[pltpu.VMEM((TM, d), jnp.float32)],
    )
    return pl.pallas_call(
        functools.partial(_ffn_body, kt_total=kt_total),
        grid_spec=grid_spec,
        out_shape=jax.ShapeDtypeStruct((n_pad, d), jnp.float32),
    )(block_expert, x_sorted, w1, b1r, w2, b2r, gate_mat)


def kernel(x, Wr, br, W1, b1, W2, b2):
    bq, t, d = x.shape
    e = Wr.shape[1]
    n = bq * t
    n_pad = ((n // TM) + e - 1) * TM
    nb_max = n_pad // TM
    x2d = x.reshape(n, d)

    top1_m, prob_m, aux_m = _run_router(x2d, Wr, br)
    top1 = top1_m[:, 0]
    prob = prob_m[:, 0]
    aux_loss = aux_m[0, 0]

    oh = (top1[:, None] == jnp.arange(e, dtype=jnp.int32)[None, :]).astype(jnp.int32)
    counts = oh.sum(axis=0)
    nb = (counts + TM - 1) // TM
    cum_nb = jnp.cumsum(nb)
    num_blocks = cum_nb[-1]
    pos_start = (cum_nb - nb) * TM
    rank = jnp.cumsum(oh, axis=0) - oh
    rank_t = jnp.take_along_axis(rank, top1[:, None], axis=1)[:, 0]
    inv_pos = pos_start[top1] + rank_t
    perm = jnp.full((n_pad,), n, jnp.int32).at[inv_pos].set(
        jnp.arange(n, dtype=jnp.int32))
    gate = jnp.zeros((n_pad,), jnp.float32).at[inv_pos].set(prob)
    gate_mat = jnp.broadcast_to(gate[:, None], (n_pad, LANES))
    block_expert = jnp.minimum(
        jnp.searchsorted(cum_nb, jnp.arange(nb_max), side="right"),
        e - 1).astype(jnp.int32)

    x_pad = jnp.concatenate([x2d, jnp.zeros((8, d), jnp.float32)], axis=0)
    x_sorted = _sc_gather(x_pad, perm, chunk=40)

    y_sorted = _run_ffn(x_sorted, W1, b1, W2, b2, gate_mat, block_expert,
                        num_blocks)

    out = _sc_gather(y_sorted, inv_pos, chunk=64)
    return out.reshape(bq, t, d), aux_loss


# speedup vs baseline: 1.5625x; 1.2451x over previous
import functools

import jax
import jax.numpy as jnp
from jax import lax
from jax.experimental import pallas as pl
from jax.experimental.pallas import tpu as pltpu
from jax.experimental.pallas import tpu_sc as plsc

TM = 256
FT = 2048
LANES = 128


def _router_body(x_ref, wr_ref, br_ref, top1_ref, prob_ref, aux_ref):
    n, e_pad = top1_ref.shape
    x = x_ref[...]
    logits = jnp.dot(x, wr_ref[...], preferred_element_type=jnp.float32,
                     precision=lax.Precision.DEFAULT) + br_ref[...]
    m = jnp.max(logits, axis=1, keepdims=True)
    p = jnp.exp(logits - m)
    s = jnp.sum(p, axis=1, keepdims=True)
    lane = lax.broadcasted_iota(jnp.int32, (n, e_pad), 1)
    top1 = jnp.min(jnp.where(logits >= m, lane, e_pad), axis=1, keepdims=True)
    top1_ref[...] = jnp.broadcast_to(top1, (n, e_pad))
    prob_ref[...] = jnp.broadcast_to(1.0 / s, (n, e_pad))
    probs = p / s
    importance = jnp.sum(probs, axis=0, keepdims=True) * (1.0 / n)
    load = jnp.sum(jnp.where(lane == top1, 1.0, 0.0), axis=0, keepdims=True) * (1.0 / n)
    e_real = aux_ref.shape[0]
    aux = e_real * jnp.sum(importance * load)
    aux_ref[...] = jnp.broadcast_to(aux, aux_ref.shape)


def _run_router(x2d, wr, br):
    n, d = x2d.shape
    e = wr.shape[1]
    wr_pad = jnp.zeros((d, LANES), jnp.float32).at[:, :e].set(wr)
    br_pad = jnp.full((1, LANES), -1e30, jnp.float32).at[0, :e].set(br)
    return pl.pallas_call(
        _router_body,
        out_shape=(
            jax.ShapeDtypeStruct((n, LANES), jnp.int32),
            jax.ShapeDtypeStruct((n, LANES), jnp.float32),
            jax.ShapeDtypeStruct((e, LANES), jnp.float32),
        ),
    )(x2d, wr_pad, br_pad)


def _sc_gather(table, idx, chunk):
    rows, d = table.shape
    b = idx.shape[0]
    info = plsc.get_sparse_core_info()
    nw = info.num_cores * info.num_subcores
    b_per_w = b // nw
    n_chunks = b_per_w // chunk
    assert b_per_w * nw == b and n_chunks * chunk == b_per_w
    mesh = plsc.VectorSubcoreMesh(core_axis_name="c", subcore_axis_name="s")

    @functools.partial(
        pl.kernel, mesh=mesh,
        out_type=jax.ShapeDtypeStruct((b, d), jnp.float32),
        scratch_types=[
            pltpu.VMEM((chunk,), jnp.int32),
            pltpu.VMEM((chunk, d), jnp.float32),
            pltpu.SemaphoreType.DMA,
        ],
    )
    def k(table_hbm, idx_hbm, out_hbm, idx_v, rows_v, sem):
        wid = lax.axis_index("s") * info.num_cores + lax.axis_index("c")
        base = wid * b_per_w
        for i in range(n_chunks):
            off = base + i * chunk
            pltpu.sync_copy(idx_hbm.at[pl.ds(off, chunk)], idx_v)
            pltpu.async_copy(table_hbm.at[idx_v], rows_v, sem).wait()
            pltpu.sync_copy(rows_v, out_hbm.at[pl.ds(off, chunk)])

    return k(table, idx)


def _ffn_body(be_ref, x_ref, w1_ref, b1_ref, w2_ref, b2_ref, gate_ref,
              out_ref, acc_ref, *, kt_total):
    kt = pl.program_id(1)
    x = x_ref[...].astype(jnp.bfloat16)
    w1 = w1_ref[0].astype(jnp.bfloat16)
    h = jnp.dot(x, w1, preferred_element_type=jnp.float32) + b1_ref[0]
    h = h * (1.0 / (1.0 + jnp.exp(-h)))
    w2 = w2_ref[0].astype(jnp.bfloat16)
    part = jnp.dot(h.astype(jnp.bfloat16), w2, preferred_element_type=jnp.float32)

    @pl.when(kt == 0)
    def _():
        acc_ref[...] = part

    @pl.when(kt > 0)
    def _():
        acc_ref[...] = acc_ref[...] + part

    @pl.when(kt == kt_total - 1)
    def _():
        g = gate_ref[:, 0:1]
        out_ref[...] = (acc_ref[...] + b2_ref[0]) * g


def _run_ffn(x_sorted, w1, b1, w2, b2, gate_mat, block_expert, num_blocks):
    n_pad, d = x_sorted.shape
    e, _, f = w1.shape
    kt_total = f // FT
    b1r = b1.reshape(e, 1, f)
    b2r = b2.reshape(e, 1, d)
    grid_spec = pltpu.PrefetchScalarGridSpec(
        num_scalar_prefetch=1,
        grid=(num_blocks, kt_total),
        in_specs=[
            pl.BlockSpec((TM, d), lambda b, kt, be: (b, 0)),
            pl.BlockSpec((1, d, FT), lambda b, kt, be: (be[b], 0, kt)),
            pl.BlockSpec((1, 1, FT), lambda b, kt, be: (be[b], 0, kt)),
            pl.BlockSpec((1, FT, d), lambda b, kt, be: (be[b], kt, 0)),
            pl.BlockSpec((1, 1, d), lambda b, kt, be: (be[b], 0, 0)),
            pl.BlockSpec((TM, LANES), lambda b, kt, be: (b, 0)),
        ],
        out_specs=pl.BlockSpec((TM, d), lambda b, kt, be: (b, 0)),
        scratch_shapes=[pltpu.VMEM((TM, d), jnp.float32)],
    )
    return pl.pallas_call(
        functools.partial(_ffn_body, kt_total=kt_total),
        grid_spec=grid_spec,
        out_shape=jax.ShapeDtypeStruct((n_pad, d), jnp.float32),
    )(block_expert, x_sorted, w1, b1r, w2, b2r, gate_mat)


def kernel(x, Wr, br, W1, b1, W2, b2):
    bq, t, d = x.shape
    e = Wr.shape[1]
    n = bq * t
    n_pad = ((n // TM) + e - 1) * TM
    nb_max = n_pad // TM
    x2d = x.reshape(n, d)

    top1_m, prob_m, aux_m = _run_router(x2d, Wr, br)
    top1 = top1_m[:, 0]
    prob = prob_m[:, 0]
    aux_loss = aux_m[0, 0]

    oh = (top1[:, None] == jnp.arange(e, dtype=jnp.int32)[None, :]).astype(jnp.int32)
    counts = oh.sum(axis=0)
    nb = (counts + TM - 1) // TM
    cum_nb = jnp.cumsum(nb)
    num_blocks = cum_nb[-1]
    pos_start = (cum_nb - nb) * TM
    rank = jnp.cumsum(oh, axis=0) - oh
    rank_t = jnp.take_along_axis(rank, top1[:, None], axis=1)[:, 0]
    inv_pos = pos_start[top1] + rank_t
    pad_fill = n + (jnp.arange(n_pad, dtype=jnp.int32) % 8)
    perm = pad_fill.at[inv_pos].set(jnp.arange(n, dtype=jnp.int32))
    gate = jnp.zeros((n_pad,), jnp.float32).at[inv_pos].set(prob)
    gate_mat = jnp.broadcast_to(gate[:, None], (n_pad, LANES))
    block_expert = jnp.minimum(
        jnp.searchsorted(cum_nb, jnp.arange(nb_max), side="right"),
        e - 1).astype(jnp.int32)

    x_pad = jnp.concatenate([x2d, jnp.zeros((8, d), jnp.float32)], axis=0)
    x_sorted = _sc_gather(x_pad, perm, chunk=120)

    y_sorted = _run_ffn(x_sorted, W1, b1, W2, b2, gate_mat, block_expert,
                        num_blocks)

    out = _sc_gather(y_sorted, inv_pos, chunk=64)
    return out.reshape(bq, t, d), aux_loss


# speedup vs baseline: 1.9776x; 1.2657x over previous
import functools

import jax
import jax.numpy as jnp
from jax import lax
from jax.experimental import pallas as pl
from jax.experimental.pallas import tpu as pltpu
from jax.experimental.pallas import tpu_sc as plsc

TM = 256
FT = 2048
LANES = 128


def _router_body(x_ref, wr_ref, br_ref,
                 prob_ref, aux_ref, pos_ref, be_ref, nb_ref, *, e, tm):
    n, e_pad = prob_ref.shape
    x = x_ref[...]
    logits = jnp.dot(x, wr_ref[...], preferred_element_type=jnp.float32,
                     precision=lax.Precision.DEFAULT) + br_ref[...]
    m = jnp.max(logits, axis=1, keepdims=True)
    p = jnp.exp(logits - m)
    s = jnp.sum(p, axis=1, keepdims=True)
    lane = lax.broadcasted_iota(jnp.int32, (n, e_pad), 1)
    top1 = jnp.min(jnp.where(logits >= m, lane, e_pad), axis=1, keepdims=True)
    prob_ref[...] = jnp.broadcast_to(1.0 / s, (n, e_pad))

    probs = p / s
    onehot = jnp.where(lane == top1, 1.0, 0.0)
    importance = jnp.sum(probs, axis=0, keepdims=True) * (1.0 / n)
    load = jnp.sum(onehot, axis=0, keepdims=True) * (1.0 / n)
    aux = e * jnp.sum(importance * load)
    aux_ref[...] = jnp.broadcast_to(aux, aux_ref.shape)

    row_i = lax.broadcasted_iota(jnp.int32, (n, n), 0)
    col_i = lax.broadcasted_iota(jnp.int32, (n, n), 1)
    tril = jnp.where(row_i > col_i, 1.0, 0.0).astype(jnp.bfloat16)
    prefix = jnp.dot(tril, onehot.astype(jnp.bfloat16),
                     preferred_element_type=jnp.float32)
    rank = jnp.sum(jnp.where(lane == top1, prefix, 0.0), axis=1, keepdims=True)

    counts = jnp.sum(onehot, axis=0, keepdims=True)
    nb = jnp.floor((counts + (tm - 1)) * (1.0 / tm))
    le_mask = jnp.where(
        lax.broadcasted_iota(jnp.int32, (e_pad, e_pad), 0)
        <= lax.broadcasted_iota(jnp.int32, (e_pad, e_pad), 1), 1.0, 0.0)
    cum_nb = jnp.dot(nb.astype(jnp.bfloat16), le_mask.astype(jnp.bfloat16),
                     preferred_element_type=jnp.float32)
    pos_start = (cum_nb - nb) * tm
    tok_start = jnp.sum(jnp.where(lane == top1, pos_start, 0.0),
                        axis=1, keepdims=True)
    pos = (tok_start + rank).astype(jnp.int32)
    pos_ref[...] = jnp.broadcast_to(pos, (n, e_pad))

    nbm = be_ref.shape[0]
    b_iota = lax.broadcasted_iota(jnp.int32, (nbm, e_pad), 0).astype(jnp.float32)
    in_e = jnp.where(lane[:nbm] < e, 1.0, 0.0)
    be = jnp.sum(jnp.where(b_iota >= jnp.broadcast_to(cum_nb, (nbm, e_pad)),
                           in_e, 0.0), axis=1, keepdims=True)
    be = jnp.minimum(be, float(e - 1)).astype(jnp.int32)
    be_ref[...] = jnp.broadcast_to(be, (nbm, e_pad))
    total = jnp.sum(nb * in_e[:1]).astype(jnp.int32)
    nb_ref[...] = jnp.broadcast_to(total, nb_ref.shape)


def _run_router(x2d, wr, br, nb_max):
    n, d = x2d.shape
    e = wr.shape[1]
    wr_pad = jnp.zeros((d, LANES), jnp.float32).at[:, :e].set(wr)
    br_pad = jnp.full((1, LANES), -1e30, jnp.float32).at[0, :e].set(br)
    return pl.pallas_call(
        functools.partial(_router_body, e=e, tm=TM),
        out_shape=(
            jax.ShapeDtypeStruct((n, LANES), jnp.float32),
            jax.ShapeDtypeStruct((e, LANES), jnp.float32),
            jax.ShapeDtypeStruct((n, LANES), jnp.int32),
            jax.ShapeDtypeStruct((nb_max, LANES), jnp.int32),
            jax.ShapeDtypeStruct((8, LANES), jnp.int32),
        ),
    )(x2d, wr_pad, br_pad)


def _sc_scatter(rows, idx, out_rows):
    b, d = rows.shape
    info = plsc.get_sparse_core_info()
    nw = info.num_cores * info.num_subcores
    b_per_w = b // nw
    assert b_per_w * nw == b and b_per_w % 8 == 0
    mesh = plsc.VectorSubcoreMesh(core_axis_name="c", subcore_axis_name="s")

    @functools.partial(
        pl.kernel, mesh=mesh,
        out_type=jax.ShapeDtypeStruct((out_rows, d), jnp.float32),
        scratch_types=[
            pltpu.VMEM((b_per_w,), jnp.int32),
            pltpu.VMEM((b_per_w, d), jnp.float32),
            pltpu.SemaphoreType.DMA,
        ],
    )
    def k(rows_hbm, idx_hbm, out_hbm, idx_v, rows_v, sem):
        wid = lax.axis_index("s") * info.num_cores + lax.axis_index("c")
        base = wid * b_per_w
        pltpu.sync_copy(idx_hbm.at[pl.ds(base, b_per_w)], idx_v)
        pltpu.sync_copy(rows_hbm.at[pl.ds(base, b_per_w)], rows_v)
        pltpu.async_copy(rows_v, out_hbm.at[idx_v], sem).wait()

    return k(rows, idx)


def _sc_gather(table, idx):
    rows, d = table.shape
    b = idx.shape[0]
    info = plsc.get_sparse_core_info()
    nw = info.num_cores * info.num_subcores
    b_per_w = b // nw
    assert b_per_w * nw == b and b_per_w % 8 == 0
    mesh = plsc.VectorSubcoreMesh(core_axis_name="c", subcore_axis_name="s")

    @functools.partial(
        pl.kernel, mesh=mesh,
        out_type=jax.ShapeDtypeStruct((b, d), jnp.float32),
        scratch_types=[
            pltpu.VMEM((b_per_w,), jnp.int32),
            pltpu.VMEM((b_per_w, d), jnp.float32),
            pltpu.SemaphoreType.DMA,
        ],
    )
    def k(table_hbm, idx_hbm, out_hbm, idx_v, rows_v, sem):
        wid = lax.axis_index("s") * info.num_cores + lax.axis_index("c")
        base = wid * b_per_w
        pltpu.sync_copy(idx_hbm.at[pl.ds(base, b_per_w)], idx_v)
        pltpu.async_copy(table_hbm.at[idx_v], rows_v, sem).wait()
        pltpu.sync_copy(rows_v, out_hbm.at[pl.ds(base, b_per_w)])

    return k(table, idx)


def _ffn_body(be_ref, x_ref, w1_ref, b1_ref, w2_ref, b2_ref,
              out_ref, acc_ref, *, kt_total):
    kt = pl.program_id(1)
    x = x_ref[...].astype(jnp.bfloat16)
    w1 = w1_ref[0].astype(jnp.bfloat16)
    h = jnp.dot(x, w1, preferred_element_type=jnp.float32) + b1_ref[0]
    h = h * (1.0 / (1.0 + jnp.exp(-h)))
    w2 = w2_ref[0].astype(jnp.bfloat16)
    part = jnp.dot(h.astype(jnp.bfloat16), w2, preferred_element_type=jnp.float32)

    @pl.when(kt == 0)
    def _():
        acc_ref[...] = part

    @pl.when(kt > 0)
    def _():
        acc_ref[...] = acc_ref[...] + part

    @pl.when(kt == kt_total - 1)
    def _():
        out_ref[...] = acc_ref[...] + b2_ref[0]


def _run_ffn(x_sorted, w1, b1, w2, b2, block_expert, num_blocks):
    n_pad, d = x_sorted.shape
    e, _, f = w1.shape
    kt_total = f // FT
    b1r = b1.reshape(e, 1, f)
    b2r = b2.reshape(e, 1, d)
    grid_spec = pltpu.PrefetchScalarGridSpec(
        num_scalar_prefetch=1,
        grid=(num_blocks, kt_total),
        in_specs=[
            pl.BlockSpec((TM, d), lambda b, kt, be: (b, 0)),
            pl.BlockSpec((1, d, FT), lambda b, kt, be: (be[b], 0, kt)),
            pl.BlockSpec((1, 1, FT), lambda b, kt, be: (be[b], 0, kt)),
            pl.BlockSpec((1, FT, d), lambda b, kt, be: (be[b], kt, 0)),
            pl.BlockSpec((1, 1, d), lambda b, kt, be: (be[b], 0, 0)),
        ],
        out_specs=pl.BlockSpec((TM, d), lambda b, kt, be: (b, 0)),
        scratch_shapes=[pltpu.VMEM((TM, d), jnp.float32)],
    )
    return pl.pallas_call(
        functools.partial(_ffn_body, kt_total=kt_total),
        grid_spec=grid_spec,
        out_shape=jax.ShapeDtypeStruct((n_pad, d), jnp.float32),
    )(block_expert, x_sorted, w1, b1r, w2, b2r)


def _gate_body(y_ref, p_ref, o_ref):
    o_ref[...] = y_ref[...] * p_ref[:, 0:1]


def _run_gate(y, prob_m):
    n, d = y.shape
    bt = 256
    return pl.pallas_call(
        _gate_body,
        grid=(n // bt,),
        in_specs=[
            pl.BlockSpec((bt, d), lambda i: (i, 0)),
            pl.BlockSpec((bt, LANES), lambda i: (i, 0)),
        ],
        out_specs=pl.BlockSpec((bt, d), lambda i: (i, 0)),
        out_shape=jax.ShapeDtypeStruct((n, d), jnp.float32),
    )(y, prob_m)


def kernel(x, Wr, br, W1, b1, W2, b2):
    bq, t, d = x.shape
    e = Wr.shape[1]
    n = bq * t
    n_pad = ((n // TM) + e - 1) * TM
    nb_max = n_pad // TM
    nb_max_pad = ((nb_max + 7) // 8) * 8
    x2d = x.reshape(n, d)

    prob_m, aux_m, pos_m, be_m, nbl_m = _run_router(x2d, Wr, br, nb_max_pad)
    aux_loss = aux_m[0, 0]
    inv_pos = pos_m[:, 0]
    block_expert = be_m[:nb_max, 0]
    num_blocks = nbl_m[0, 0]

    x_sorted = _sc_scatter(x2d, inv_pos, n_pad)
    y_sorted = _run_ffn(x_sorted, W1, b1, W2, b2, block_expert, num_blocks)
    y_tok = _sc_gather(y_sorted, inv_pos)
    out = _run_gate(y_tok, prob_m)
    return out.reshape(bq, t, d), aux_loss


# speedup vs baseline: 2.4619x; 1.2449x over previous
import functools

import jax
import jax.numpy as jnp
from jax import lax
from jax.experimental import pallas as pl
from jax.experimental.pallas import tpu as pltpu
from jax.experimental.pallas import tpu_sc as plsc

TM = 512
FT = 2048
LANES = 128


def _router_body(x_ref, wr_ref, br_ref,
                 prob_ref, aux_ref, pos_ref, be_ref, nb_ref, *, e, tm):
    n, e_pad = prob_ref.shape
    x = x_ref[...]
    logits = jnp.dot(x, wr_ref[...], preferred_element_type=jnp.float32,
                     precision=lax.Precision.DEFAULT) + br_ref[...]
    m = jnp.max(logits, axis=1, keepdims=True)
    p = jnp.exp(logits - m)
    s = jnp.sum(p, axis=1, keepdims=True)
    lane = lax.broadcasted_iota(jnp.int32, (n, e_pad), 1)
    top1 = jnp.min(jnp.where(logits >= m, lane, e_pad), axis=1, keepdims=True)
    prob_ref[...] = jnp.broadcast_to(1.0 / s, (n, e_pad))

    probs = p / s
    onehot = jnp.where(lane == top1, 1.0, 0.0)
    importance = jnp.sum(probs, axis=0, keepdims=True) * (1.0 / n)
    load = jnp.sum(onehot, axis=0, keepdims=True) * (1.0 / n)
    aux = e * jnp.sum(importance * load)
    aux_ref[...] = jnp.broadcast_to(aux, aux_ref.shape)

    row_i = lax.broadcasted_iota(jnp.int32, (n, n), 0)
    col_i = lax.broadcasted_iota(jnp.int32, (n, n), 1)
    tril = jnp.where(row_i > col_i, 1.0, 0.0).astype(jnp.bfloat16)
    prefix = jnp.dot(tril, onehot.astype(jnp.bfloat16),
                     preferred_element_type=jnp.float32)
    rank = jnp.sum(jnp.where(lane == top1, prefix, 0.0), axis=1, keepdims=True)

    counts = jnp.sum(onehot, axis=0, keepdims=True)
    nb = jnp.floor((counts + (tm - 1)) * (1.0 / tm))
    le_mask = jnp.where(
        lax.broadcasted_iota(jnp.int32, (e_pad, e_pad), 0)
        <= lax.broadcasted_iota(jnp.int32, (e_pad, e_pad), 1), 1.0, 0.0)
    cum_nb = jnp.dot(nb.astype(jnp.bfloat16), le_mask.astype(jnp.bfloat16),
                     preferred_element_type=jnp.float32)
    pos_start = (cum_nb - nb) * tm
    tok_start = jnp.sum(jnp.where(lane == top1, pos_start, 0.0),
                        axis=1, keepdims=True)
    pos = (tok_start + rank).astype(jnp.int32)
    pos_ref[...] = jnp.broadcast_to(pos, (n, e_pad))

    nbm = be_ref.shape[0]
    b_iota = lax.broadcasted_iota(jnp.int32, (nbm, e_pad), 0).astype(jnp.float32)
    in_e = jnp.where(lane[:nbm] < e, 1.0, 0.0)
    be = jnp.sum(jnp.where(b_iota >= jnp.broadcast_to(cum_nb, (nbm, e_pad)),
                           in_e, 0.0), axis=1, keepdims=True)
    be = jnp.minimum(be, float(e - 1)).astype(jnp.int32)
    be_ref[...] = jnp.broadcast_to(be, (nbm, e_pad))
    total = jnp.sum(nb * in_e[:1]).astype(jnp.int32)
    nb_ref[...] = jnp.broadcast_to(total, nb_ref.shape)


def _run_router(x2d, wr, br, nb_max):
    n, d = x2d.shape
    e = wr.shape[1]
    wr_pad = jnp.zeros((d, LANES), jnp.float32).at[:, :e].set(wr)
    br_pad = jnp.full((1, LANES), -1e30, jnp.float32).at[0, :e].set(br)
    return pl.pallas_call(
        functools.partial(_router_body, e=e, tm=TM),
        out_shape=(
            jax.ShapeDtypeStruct((n, LANES), jnp.float32),
            jax.ShapeDtypeStruct((e, LANES), jnp.float32),
            jax.ShapeDtypeStruct((n, LANES), jnp.int32),
            jax.ShapeDtypeStruct((nb_max, LANES), jnp.int32),
            jax.ShapeDtypeStruct((8, LANES), jnp.int32),
        ),
    )(x2d, wr_pad, br_pad)


def _sc_scatter(rows, idx, out_rows):
    b, d = rows.shape
    info = plsc.get_sparse_core_info()
    nw = info.num_cores * info.num_subcores
    b_per_w = b // nw
    assert b_per_w * nw == b and b_per_w % 8 == 0
    mesh = plsc.VectorSubcoreMesh(core_axis_name="c", subcore_axis_name="s")

    @functools.partial(
        pl.kernel, mesh=mesh,
        out_type=jax.ShapeDtypeStruct((out_rows, d), jnp.float32),
        scratch_types=[
            pltpu.VMEM((b_per_w,), jnp.int32),
            pltpu.VMEM((b_per_w, d), jnp.float32),
            pltpu.SemaphoreType.DMA,
        ],
    )
    def k(rows_hbm, idx_hbm, out_hbm, idx_v, rows_v, sem):
        wid = lax.axis_index("s") * info.num_cores + lax.axis_index("c")
        base = wid * b_per_w
        pltpu.sync_copy(idx_hbm.at[pl.ds(base, b_per_w)], idx_v)
        pltpu.sync_copy(rows_hbm.at[pl.ds(base, b_per_w)], rows_v)
        pltpu.async_copy(rows_v, out_hbm.at[idx_v], sem).wait()

    return k(rows, idx)


def _sc_gather(table, idx):
    rows, d = table.shape
    b = idx.shape[0]
    info = plsc.get_sparse_core_info()
    nw = info.num_cores * info.num_subcores
    b_per_w = b // nw
    assert b_per_w * nw == b and b_per_w % 8 == 0
    mesh = plsc.VectorSubcoreMesh(core_axis_name="c", subcore_axis_name="s")

    @functools.partial(
        pl.kernel, mesh=mesh,
        out_type=jax.ShapeDtypeStruct((b, d), jnp.float32),
        scratch_types=[
            pltpu.VMEM((b_per_w,), jnp.int32),
            pltpu.VMEM((b_per_w, d), jnp.float32),
            pltpu.SemaphoreType.DMA,
        ],
    )
    def k(table_hbm, idx_hbm, out_hbm, idx_v, rows_v, sem):
        wid = lax.axis_index("s") * info.num_cores + lax.axis_index("c")
        base = wid * b_per_w
        pltpu.sync_copy(idx_hbm.at[pl.ds(base, b_per_w)], idx_v)
        pltpu.async_copy(table_hbm.at[idx_v], rows_v, sem).wait()
        pltpu.sync_copy(rows_v, out_hbm.at[pl.ds(base, b_per_w)])

    return k(table, idx)


def _ffn_body(be_ref, x_ref, w1_ref, b1_ref, w2_ref, b2_ref,
              out_ref, acc_ref, *, kt_total):
    kt = pl.program_id(1)
    x = x_ref[...].astype(jnp.bfloat16)
    w1 = w1_ref[0].astype(jnp.bfloat16)
    h = jnp.dot(x, w1, preferred_element_type=jnp.float32) + b1_ref[0]
    h = h * (1.0 / (1.0 + jnp.exp(-h)))
    w2 = w2_ref[0].astype(jnp.bfloat16)
    part = jnp.dot(h.astype(jnp.bfloat16), w2, preferred_element_type=jnp.float32)

    @pl.when(kt == 0)
    def _():
        acc_ref[...] = part

    @pl.when(kt > 0)
    def _():
        acc_ref[...] = acc_ref[...] + part

    @pl.when(kt == kt_total - 1)
    def _():
        out_ref[...] = acc_ref[...] + b2_ref[0]


def _run_ffn(x_sorted, w1, b1, w2, b2, block_expert, num_blocks):
    n_pad, d = x_sorted.shape
    e, _, f = w1.shape
    kt_total = f // FT
    b1r = b1.reshape(e, 1, f)
    b2r = b2.reshape(e, 1, d)
    grid_spec = pltpu.PrefetchScalarGridSpec(
        num_scalar_prefetch=1,
        grid=(num_blocks, kt_total),
        in_specs=[
            pl.BlockSpec((TM, d), lambda b, kt, be: (b, 0)),
            pl.BlockSpec((1, d, FT), lambda b, kt, be: (be[b], 0, kt)),
            pl.BlockSpec((1, 1, FT), lambda b, kt, be: (be[b], 0, kt)),
            pl.BlockSpec((1, FT, d), lambda b, kt, be: (be[b], kt, 0)),
            pl.BlockSpec((1, 1, d), lambda b, kt, be: (be[b], 0, 0)),
        ],
        out_specs=pl.BlockSpec((TM, d), lambda b, kt, be: (b, 0)),
        scratch_shapes=[pltpu.VMEM((TM, d), jnp.float32)],
    )
    return pl.pallas_call(
        functools.partial(_ffn_body, kt_total=kt_total),
        grid_spec=grid_spec,
        out_shape=jax.ShapeDtypeStruct((n_pad, d), jnp.float32),
    )(block_expert, x_sorted, w1, b1r, w2, b2r)


def _gate_body(y_ref, p_ref, o_ref):
    o_ref[...] = y_ref[...] * p_ref[:, 0:1]


def _run_gate(y, prob_m):
    n, d = y.shape
    bt = 256
    return pl.pallas_call(
        _gate_body,
        grid=(n // bt,),
        in_specs=[
            pl.BlockSpec((bt, d), lambda i: (i, 0)),
            pl.BlockSpec((bt, LANES), lambda i: (i, 0)),
        ],
        out_specs=pl.BlockSpec((bt, d), lambda i: (i, 0)),
        out_shape=jax.ShapeDtypeStruct((n, d), jnp.float32),
    )(y, prob_m)


def kernel(x, Wr, br, W1, b1, W2, b2):
    bq, t, d = x.shape
    e = Wr.shape[1]
    n = bq * t
    n_pad = (((n + TM - 1) // TM) + e - 1) * TM
    nb_max = n_pad // TM
    nb_max_pad = ((nb_max + 7) // 8) * 8
    x2d = x.reshape(n, d)

    prob_m, aux_m, pos_m, be_m, nbl_m = _run_router(x2d, Wr, br, nb_max_pad)
    aux_loss = aux_m[0, 0]
    inv_pos = pos_m[:, 0]
    block_expert = be_m[:nb_max, 0]
    num_blocks = nbl_m[0, 0]

    x_sorted = _sc_scatter(x2d, inv_pos, n_pad)
    y_sorted = _run_ffn(x_sorted, W1, b1, W2, b2, block_expert, num_blocks)
    y_tok = _sc_gather(y_sorted, inv_pos)
    out = _run_gate(y_tok, prob_m)
    return out.reshape(bq, t, d), aux_loss


# speedup vs baseline: 2.5937x; 1.0535x over previous
import functools

import jax
import jax.numpy as jnp
from jax import lax
from jax.experimental import pallas as pl
from jax.experimental.pallas import tpu as pltpu
from jax.experimental.pallas import tpu_sc as plsc

TM = 384
FT = 2048
LANES = 128


def _router_body(x_ref, wr_ref, br_ref,
                 prob_ref, aux_ref, pos_ref, be_ref, nb_ref, *, e, tm):
    n, e_pad = prob_ref.shape
    x = x_ref[...]
    logits = jnp.dot(x, wr_ref[...], preferred_element_type=jnp.float32,
                     precision=lax.Precision.DEFAULT) + br_ref[...]
    m = jnp.max(logits, axis=1, keepdims=True)
    p = jnp.exp(logits - m)
    s = jnp.sum(p, axis=1, keepdims=True)
    lane = lax.broadcasted_iota(jnp.int32, (n, e_pad), 1)
    top1 = jnp.min(jnp.where(logits >= m, lane, e_pad), axis=1, keepdims=True)
    prob_ref[...] = jnp.broadcast_to(1.0 / s, (n, e_pad))

    probs = p / s
    onehot = jnp.where(lane == top1, 1.0, 0.0)
    importance = jnp.sum(probs, axis=0, keepdims=True) * (1.0 / n)
    load = jnp.sum(onehot, axis=0, keepdims=True) * (1.0 / n)
    aux = e * jnp.sum(importance * load)
    aux_ref[...] = jnp.broadcast_to(aux, aux_ref.shape)

    row_i = lax.broadcasted_iota(jnp.int32, (n, n), 0)
    col_i = lax.broadcasted_iota(jnp.int32, (n, n), 1)
    tril = jnp.where(row_i > col_i, 1.0, 0.0).astype(jnp.bfloat16)
    prefix = jnp.dot(tril, onehot.astype(jnp.bfloat16),
                     preferred_element_type=jnp.float32)
    rank = jnp.sum(jnp.where(lane == top1, prefix, 0.0), axis=1, keepdims=True)

    counts = jnp.sum(onehot, axis=0, keepdims=True)
    nb = jnp.floor((counts + (tm - 1)) * (1.0 / tm))
    le_mask = jnp.where(
        lax.broadcasted_iota(jnp.int32, (e_pad, e_pad), 0)
        <= lax.broadcasted_iota(jnp.int32, (e_pad, e_pad), 1), 1.0, 0.0)
    cum_nb = jnp.dot(nb.astype(jnp.bfloat16), le_mask.astype(jnp.bfloat16),
                     preferred_element_type=jnp.float32)
    pos_start = (cum_nb - nb) * tm
    tok_start = jnp.sum(jnp.where(lane == top1, pos_start, 0.0),
                        axis=1, keepdims=True)
    pos = (tok_start + rank).astype(jnp.int32)
    pos_ref[...] = jnp.broadcast_to(pos, (n, e_pad))

    nbm = be_ref.shape[0]
    b_iota = lax.broadcasted_iota(jnp.int32, (nbm, e_pad), 0).astype(jnp.float32)
    in_e = jnp.where(lane[:nbm] < e, 1.0, 0.0)
    be = jnp.sum(jnp.where(b_iota >= jnp.broadcast_to(cum_nb, (nbm, e_pad)),
                           in_e, 0.0), axis=1, keepdims=True)
    be = jnp.minimum(be, float(e - 1)).astype(jnp.int32)
    be_ref[...] = jnp.broadcast_to(be, (nbm, e_pad))
    total = jnp.sum(nb * in_e[:1]).astype(jnp.int32)
    nb_ref[...] = jnp.broadcast_to(total, nb_ref.shape)


def _run_router(x2d, wr, br, nb_max):
    n, d = x2d.shape
    e = wr.shape[1]
    wr_pad = jnp.zeros((d, LANES), jnp.float32).at[:, :e].set(wr)
    br_pad = jnp.full((1, LANES), -1e30, jnp.float32).at[0, :e].set(br)
    return pl.pallas_call(
        functools.partial(_router_body, e=e, tm=TM),
        out_shape=(
            jax.ShapeDtypeStruct((n, LANES), jnp.float32),
            jax.ShapeDtypeStruct((e, LANES), jnp.float32),
            jax.ShapeDtypeStruct((n, LANES), jnp.int32),
            jax.ShapeDtypeStruct((nb_max, LANES), jnp.int32),
            jax.ShapeDtypeStruct((8, LANES), jnp.int32),
        ),
    )(x2d, wr_pad, br_pad)


def _sc_scatter(rows, idx, out_rows):
    b, d = rows.shape
    info = plsc.get_sparse_core_info()
    nw = info.num_cores * info.num_subcores
    b_per_w = b // nw
    assert b_per_w * nw == b and b_per_w % 8 == 0
    mesh = plsc.VectorSubcoreMesh(core_axis_name="c", subcore_axis_name="s")

    @functools.partial(
        pl.kernel, mesh=mesh,
        out_type=jax.ShapeDtypeStruct((out_rows, d), jnp.float32),
        scratch_types=[
            pltpu.VMEM((b_per_w,), jnp.int32),
            pltpu.VMEM((b_per_w, d), jnp.float32),
            pltpu.SemaphoreType.DMA,
        ],
    )
    def k(rows_hbm, idx_hbm, out_hbm, idx_v, rows_v, sem):
        wid = lax.axis_index("s") * info.num_cores + lax.axis_index("c")
        base = wid * b_per_w
        pltpu.sync_copy(idx_hbm.at[pl.ds(base, b_per_w)], idx_v)
        pltpu.sync_copy(rows_hbm.at[pl.ds(base, b_per_w)], rows_v)
        pltpu.async_copy(rows_v, out_hbm.at[idx_v], sem).wait()

    return k(rows, idx)


def _sc_gather(table, idx):
    rows, d = table.shape
    b = idx.shape[0]
    info = plsc.get_sparse_core_info()
    nw = info.num_cores * info.num_subcores
    b_per_w = b // nw
    assert b_per_w * nw == b and b_per_w % 8 == 0
    mesh = plsc.VectorSubcoreMesh(core_axis_name="c", subcore_axis_name="s")

    @functools.partial(
        pl.kernel, mesh=mesh,
        out_type=jax.ShapeDtypeStruct((b, d), jnp.float32),
        scratch_types=[
            pltpu.VMEM((b_per_w,), jnp.int32),
            pltpu.VMEM((b_per_w, d), jnp.float32),
            pltpu.SemaphoreType.DMA,
        ],
    )
    def k(table_hbm, idx_hbm, out_hbm, idx_v, rows_v, sem):
        wid = lax.axis_index("s") * info.num_cores + lax.axis_index("c")
        base = wid * b_per_w
        pltpu.sync_copy(idx_hbm.at[pl.ds(base, b_per_w)], idx_v)
        pltpu.async_copy(table_hbm.at[idx_v], rows_v, sem).wait()
        pltpu.sync_copy(rows_v, out_hbm.at[pl.ds(base, b_per_w)])

    return k(table, idx)


def _ffn_body(be_ref, x_ref, w1_ref, b1_ref, w2_ref, b2_ref,
              out_ref, acc_ref, *, kt_total):
    kt = pl.program_id(1)
    x = x_ref[...].astype(jnp.bfloat16)
    w1 = w1_ref[0].astype(jnp.bfloat16)
    h = jnp.dot(x, w1, preferred_element_type=jnp.float32) + b1_ref[0]
    h = h * (1.0 / (1.0 + jnp.exp(-h)))
    w2 = w2_ref[0].astype(jnp.bfloat16)
    part = jnp.dot(h.astype(jnp.bfloat16), w2, preferred_element_type=jnp.float32)

    @pl.when(kt == 0)
    def _():
        acc_ref[...] = part

    @pl.when(kt > 0)
    def _():
        acc_ref[...] = acc_ref[...] + part

    @pl.when(kt == kt_total - 1)
    def _():
        out_ref[...] = acc_ref[...] + b2_ref[0]


def _run_ffn(x_sorted, w1, b1, w2, b2, block_expert, num_blocks):
    n_pad, d = x_sorted.shape
    e, _, f = w1.shape
    kt_total = f // FT
    b1r = b1.reshape(e, 1, f)
    b2r = b2.reshape(e, 1, d)
    grid_spec = pltpu.PrefetchScalarGridSpec(
        num_scalar_prefetch=1,
        grid=(num_blocks, kt_total),
        in_specs=[
            pl.BlockSpec((TM, d), lambda b, kt, be: (b, 0)),
            pl.BlockSpec((1, d, FT), lambda b, kt, be: (be[b], 0, kt)),
            pl.BlockSpec((1, 1, FT), lambda b, kt, be: (be[b], 0, kt)),
            pl.BlockSpec((1, FT, d), lambda b, kt, be: (be[b], kt, 0)),
            pl.BlockSpec((1, 1, d), lambda b, kt, be: (be[b], 0, 0)),
        ],
        out_specs=pl.BlockSpec((TM, d), lambda b, kt, be: (b, 0)),
        scratch_shapes=[pltpu.VMEM((TM, d), jnp.float32)],
    )
    return pl.pallas_call(
        functools.partial(_ffn_body, kt_total=kt_total),
        grid_spec=grid_spec,
        out_shape=jax.ShapeDtypeStruct((n_pad, d), jnp.float32),
    )(block_expert, x_sorted, w1, b1r, w2, b2r)


def _gate_body(y_ref, p_ref, o_ref):
    o_ref[...] = y_ref[...] * p_ref[:, 0:1]


def _run_gate(y, prob_m):
    n, d = y.shape
    bt = 256
    return pl.pallas_call(
        _gate_body,
        grid=(n // bt,),
        in_specs=[
            pl.BlockSpec((bt, d), lambda i: (i, 0)),
            pl.BlockSpec((bt, LANES), lambda i: (i, 0)),
        ],
        out_specs=pl.BlockSpec((bt, d), lambda i: (i, 0)),
        out_shape=jax.ShapeDtypeStruct((n, d), jnp.float32),
    )(y, prob_m)


def kernel(x, Wr, br, W1, b1, W2, b2):
    bq, t, d = x.shape
    e = Wr.shape[1]
    n = bq * t
    n_pad = (((n + TM - 1) // TM) + e - 1) * TM
    nb_max = n_pad // TM
    nb_max_pad = ((nb_max + 7) // 8) * 8
    x2d = x.reshape(n, d)

    prob_m, aux_m, pos_m, be_m, nbl_m = _run_router(x2d, Wr, br, nb_max_pad)
    aux_loss = aux_m[0, 0]
    inv_pos = pos_m[:, 0]
    block_expert = be_m[:nb_max, 0]
    num_blocks = nbl_m[0, 0]

    x_sorted = _sc_scatter(x2d, inv_pos, n_pad)
    y_sorted = _run_ffn(x_sorted, W1, b1, W2, b2, block_expert, num_blocks)
    y_tok = _sc_gather(y_sorted, inv_pos)
    out = _run_gate(y_tok, prob_m)
    return out.reshape(bq, t, d), aux_loss


# speedup vs baseline: 2.6347x; 1.0158x over previous
import functools

import jax
import jax.numpy as jnp
from jax import lax
from jax.experimental import pallas as pl
from jax.experimental.pallas import tpu as pltpu
from jax.experimental.pallas import tpu_sc as plsc

TM = 320
FT = 2048
LANES = 128


def _router_body(x_ref, wr_ref, br_ref,
                 prob_ref, aux_ref, pos_ref, be_ref, nb_ref, *, e, tm):
    n, e_pad = prob_ref.shape
    x = x_ref[...]
    logits = jnp.dot(x, wr_ref[...], preferred_element_type=jnp.float32,
                     precision=lax.Precision.DEFAULT) + br_ref[...]
    m = jnp.max(logits, axis=1, keepdims=True)
    p = jnp.exp(logits - m)
    s = jnp.sum(p, axis=1, keepdims=True)
    lane = lax.broadcasted_iota(jnp.int32, (n, e_pad), 1)
    top1 = jnp.min(jnp.where(logits >= m, lane, e_pad), axis=1, keepdims=True)
    prob_ref[...] = jnp.broadcast_to(1.0 / s, (n, e_pad))

    probs = p / s
    onehot = jnp.where(lane == top1, 1.0, 0.0)
    importance = jnp.sum(probs, axis=0, keepdims=True) * (1.0 / n)
    load = jnp.sum(onehot, axis=0, keepdims=True) * (1.0 / n)
    aux = e * jnp.sum(importance * load)
    aux_ref[...] = jnp.broadcast_to(aux, aux_ref.shape)

    row_i = lax.broadcasted_iota(jnp.int32, (n, n), 0)
    col_i = lax.broadcasted_iota(jnp.int32, (n, n), 1)
    tril = jnp.where(row_i > col_i, 1.0, 0.0).astype(jnp.bfloat16)
    prefix = jnp.dot(tril, onehot.astype(jnp.bfloat16),
                     preferred_element_type=jnp.float32)
    rank = jnp.sum(jnp.where(lane == top1, prefix, 0.0), axis=1, keepdims=True)

    counts = jnp.sum(onehot, axis=0, keepdims=True)
    nb = jnp.floor((counts + (tm - 1)) * (1.0 / tm))
    le_mask = jnp.where(
        lax.broadcasted_iota(jnp.int32, (e_pad, e_pad), 0)
        <= lax.broadcasted_iota(jnp.int32, (e_pad, e_pad), 1), 1.0, 0.0)
    cum_nb = jnp.dot(nb.astype(jnp.bfloat16), le_mask.astype(jnp.bfloat16),
                     preferred_element_type=jnp.float32)
    pos_start = (cum_nb - nb) * tm
    tok_start = jnp.sum(jnp.where(lane == top1, pos_start, 0.0),
                        axis=1, keepdims=True)
    pos = (tok_start + rank).astype(jnp.int32)
    pos_ref[...] = jnp.broadcast_to(pos, (n, e_pad))

    nbm = be_ref.shape[0]
    b_iota = lax.broadcasted_iota(jnp.int32, (nbm, e_pad), 0).astype(jnp.float32)
    in_e = jnp.where(lane[:nbm] < e, 1.0, 0.0)
    be = jnp.sum(jnp.where(b_iota >= jnp.broadcast_to(cum_nb, (nbm, e_pad)),
                           in_e, 0.0), axis=1, keepdims=True)
    be = jnp.minimum(be, float(e - 1)).astype(jnp.int32)
    be_ref[...] = jnp.broadcast_to(be, (nbm, e_pad))
    total = jnp.sum(nb * in_e[:1]).astype(jnp.int32)
    nb_ref[...] = jnp.broadcast_to(total, nb_ref.shape)


def _run_router(x2d, wr, br, nb_max):
    n, d = x2d.shape
    e = wr.shape[1]
    wr_pad = jnp.zeros((d, LANES), jnp.float32).at[:, :e].set(wr)
    br_pad = jnp.full((1, LANES), -1e30, jnp.float32).at[0, :e].set(br)
    return pl.pallas_call(
        functools.partial(_router_body, e=e, tm=TM),
        out_shape=(
            jax.ShapeDtypeStruct((n, LANES), jnp.float32),
            jax.ShapeDtypeStruct((e, LANES), jnp.float32),
            jax.ShapeDtypeStruct((n, LANES), jnp.int32),
            jax.ShapeDtypeStruct((nb_max, LANES), jnp.int32),
            jax.ShapeDtypeStruct((8, LANES), jnp.int32),
        ),
    )(x2d, wr_pad, br_pad)


def _sc_scatter(rows, idx, out_rows):
    b, d = rows.shape
    info = plsc.get_sparse_core_info()
    nw = info.num_cores * info.num_subcores
    b_per_w = b // nw
    assert b_per_w * nw == b and b_per_w % 8 == 0
    mesh = plsc.VectorSubcoreMesh(core_axis_name="c", subcore_axis_name="s")

    @functools.partial(
        pl.kernel, mesh=mesh,
        out_type=jax.ShapeDtypeStruct((out_rows, d), jnp.float32),
        scratch_types=[
            pltpu.VMEM((b_per_w,), jnp.int32),
            pltpu.VMEM((b_per_w, d), jnp.float32),
            pltpu.SemaphoreType.DMA,
        ],
    )
    def k(rows_hbm, idx_hbm, out_hbm, idx_v, rows_v, sem):
        wid = lax.axis_index("s") * info.num_cores + lax.axis_index("c")
        base = wid * b_per_w
        pltpu.sync_copy(idx_hbm.at[pl.ds(base, b_per_w)], idx_v)
        pltpu.sync_copy(rows_hbm.at[pl.ds(base, b_per_w)], rows_v)
        pltpu.async_copy(rows_v, out_hbm.at[idx_v], sem).wait()

    return k(rows, idx)


def _sc_gather(table, idx):
    rows, d = table.shape
    b = idx.shape[0]
    info = plsc.get_sparse_core_info()
    nw = info.num_cores * info.num_subcores
    b_per_w = b // nw
    assert b_per_w * nw == b and b_per_w % 8 == 0
    mesh = plsc.VectorSubcoreMesh(core_axis_name="c", subcore_axis_name="s")

    @functools.partial(
        pl.kernel, mesh=mesh,
        out_type=jax.ShapeDtypeStruct((b, d), jnp.float32),
        scratch_types=[
            pltpu.VMEM((b_per_w,), jnp.int32),
            pltpu.VMEM((b_per_w, d), jnp.float32),
            pltpu.SemaphoreType.DMA,
        ],
    )
    def k(table_hbm, idx_hbm, out_hbm, idx_v, rows_v, sem):
        wid = lax.axis_index("s") * info.num_cores + lax.axis_index("c")
        base = wid * b_per_w
        pltpu.sync_copy(idx_hbm.at[pl.ds(base, b_per_w)], idx_v)
        pltpu.async_copy(table_hbm.at[idx_v], rows_v, sem).wait()
        pltpu.sync_copy(rows_v, out_hbm.at[pl.ds(base, b_per_w)])

    return k(table, idx)


def _ffn_body(be_ref, x_ref, w1_ref, b1_ref, w2_ref, b2_ref,
              out_ref, acc_ref, *, kt_total):
    kt = pl.program_id(1)
    x = x_ref[...].astype(jnp.bfloat16)
    w1 = w1_ref[0].astype(jnp.bfloat16)
    h = jnp.dot(x, w1, preferred_element_type=jnp.float32) + b1_ref[0]
    h = h * (1.0 / (1.0 + jnp.exp(-h)))
    w2 = w2_ref[0].astype(jnp.bfloat16)
    part = jnp.dot(h.astype(jnp.bfloat16), w2, preferred_element_type=jnp.float32)

    @pl.when(kt == 0)
    def _():
        acc_ref[...] = part

    @pl.when(kt > 0)
    def _():
        acc_ref[...] = acc_ref[...] + part

    @pl.when(kt == kt_total - 1)
    def _():
        out_ref[...] = acc_ref[...] + b2_ref[0]


def _run_ffn(x_sorted, w1, b1, w2, b2, block_expert, num_blocks):
    n_pad, d = x_sorted.shape
    e, _, f = w1.shape
    kt_total = f // FT
    b1r = b1.reshape(e, 1, f)
    b2r = b2.reshape(e, 1, d)
    grid_spec = pltpu.PrefetchScalarGridSpec(
        num_scalar_prefetch=1,
        grid=(num_blocks, kt_total),
        in_specs=[
            pl.BlockSpec((TM, d), lambda b, kt, be: (b, 0)),
            pl.BlockSpec((1, d, FT), lambda b, kt, be: (be[b], 0, kt)),
            pl.BlockSpec((1, 1, FT), lambda b, kt, be: (be[b], 0, kt)),
            pl.BlockSpec((1, FT, d), lambda b, kt, be: (be[b], kt, 0)),
            pl.BlockSpec((1, 1, d), lambda b, kt, be: (be[b], 0, 0)),
        ],
        out_specs=pl.BlockSpec((TM, d), lambda b, kt, be: (b, 0)),
        scratch_shapes=[pltpu.VMEM((TM, d), jnp.float32)],
    )
    return pl.pallas_call(
        functools.partial(_ffn_body, kt_total=kt_total),
        grid_spec=grid_spec,
        out_shape=jax.ShapeDtypeStruct((n_pad, d), jnp.float32),
    )(block_expert, x_sorted, w1, b1r, w2, b2r)


def _gate_body(y_ref, p_ref, o_ref):
    o_ref[...] = y_ref[...] * p_ref[:, 0:1]


def _run_gate(y, prob_m):
    n, d = y.shape
    bt = 256
    return pl.pallas_call(
        _gate_body,
        grid=(n // bt,),
        in_specs=[
            pl.BlockSpec((bt, d), lambda i: (i, 0)),
            pl.BlockSpec((bt, LANES), lambda i: (i, 0)),
        ],
        out_specs=pl.BlockSpec((bt, d), lambda i: (i, 0)),
        out_shape=jax.ShapeDtypeStruct((n, d), jnp.float32),
    )(y, prob_m)


def kernel(x, Wr, br, W1, b1, W2, b2):
    bq, t, d = x.shape
    e = Wr.shape[1]
    n = bq * t
    n_pad = (((n + TM - 1) // TM) + e - 1) * TM
    nb_max = n_pad // TM
    nb_max_pad = ((nb_max + 7) // 8) * 8
    x2d = x.reshape(n, d)

    prob_m, aux_m, pos_m, be_m, nbl_m = _run_router(x2d, Wr, br, nb_max_pad)
    aux_loss = aux_m[0, 0]
    inv_pos = pos_m[:, 0]
    block_expert = be_m[:nb_max, 0]
    num_blocks = nbl_m[0, 0]

    x_sorted = _sc_scatter(x2d, inv_pos, n_pad)
    y_sorted = _run_ffn(x_sorted, W1, b1, W2, b2, block_expert, num_blocks)
    y_tok = _sc_gather(y_sorted, inv_pos)
    out = _run_gate(y_tok, prob_m)
    return out.reshape(bq, t, d), aux_loss


# speedup vs baseline: 2.7919x; 1.0597x over previous
import functools

import jax
import jax.numpy as jnp
from jax import lax
from jax.experimental import pallas as pl
from jax.experimental.pallas import tpu as pltpu
from jax.experimental.pallas import tpu_sc as plsc

TM = 320
FT = 2048
LANES = 128


def _router_body(x_ref, wr_ref, br_ref,
                 prob_ref, aux_ref, pos_ref, be_ref, nb_ref, *, e, tm):
    n, e_pad = prob_ref.shape
    x = x_ref[...]
    logits = jnp.dot(x, wr_ref[...], preferred_element_type=jnp.float32,
                     precision=lax.Precision.DEFAULT) + br_ref[...]
    m = jnp.max(logits, axis=1, keepdims=True)
    p = jnp.exp(logits - m)
    s = jnp.sum(p, axis=1, keepdims=True)
    lane = lax.broadcasted_iota(jnp.int32, (n, e_pad), 1)
    top1 = jnp.min(jnp.where(logits >= m, lane, e_pad), axis=1, keepdims=True)
    prob_ref[...] = jnp.broadcast_to(1.0 / s, (n, e_pad))

    probs = p / s
    onehot = jnp.where(lane == top1, 1.0, 0.0)
    importance = jnp.sum(probs, axis=0, keepdims=True) * (1.0 / n)
    load = jnp.sum(onehot, axis=0, keepdims=True) * (1.0 / n)
    aux = e * jnp.sum(importance * load)
    aux_ref[...] = jnp.broadcast_to(aux, aux_ref.shape)

    row_i = lax.broadcasted_iota(jnp.int32, (n, n), 0)
    col_i = lax.broadcasted_iota(jnp.int32, (n, n), 1)
    tril = jnp.where(row_i > col_i, 1.0, 0.0).astype(jnp.bfloat16)
    prefix = jnp.dot(tril, onehot.astype(jnp.bfloat16),
                     preferred_element_type=jnp.float32)
    rank = jnp.sum(jnp.where(lane == top1, prefix, 0.0), axis=1, keepdims=True)

    counts = jnp.sum(onehot, axis=0, keepdims=True)
    nb = jnp.floor((counts + (tm - 1)) * (1.0 / tm))
    le_mask = jnp.where(
        lax.broadcasted_iota(jnp.int32, (e_pad, e_pad), 0)
        <= lax.broadcasted_iota(jnp.int32, (e_pad, e_pad), 1), 1.0, 0.0)
    cum_nb = jnp.dot(nb.astype(jnp.bfloat16), le_mask.astype(jnp.bfloat16),
                     preferred_element_type=jnp.float32)
    pos_start = (cum_nb - nb) * tm
    tok_start = jnp.sum(jnp.where(lane == top1, pos_start, 0.0),
                        axis=1, keepdims=True)
    pos = (tok_start + rank).astype(jnp.int32)
    pos_ref[...] = jnp.broadcast_to(pos, (n, e_pad))

    nbm = be_ref.shape[0]
    b_iota = lax.broadcasted_iota(jnp.int32, (nbm, e_pad), 0).astype(jnp.float32)
    in_e = jnp.where(lane[:nbm] < e, 1.0, 0.0)
    be = jnp.sum(jnp.where(b_iota >= jnp.broadcast_to(cum_nb, (nbm, e_pad)),
                           in_e, 0.0), axis=1, keepdims=True)
    be = jnp.minimum(be, float(e - 1)).astype(jnp.int32)
    be_ref[...] = jnp.broadcast_to(be, (nbm, e_pad))
    total = jnp.sum(nb * in_e[:1]).astype(jnp.int32)
    nb_ref[...] = jnp.broadcast_to(total, nb_ref.shape)


def _run_router(x2d, wr, br, nb_max):
    n, d = x2d.shape
    e = wr.shape[1]
    wr_pad = jnp.zeros((d, LANES), jnp.float32).at[:, :e].set(wr)
    br_pad = jnp.full((1, LANES), -1e30, jnp.float32).at[0, :e].set(br)
    return pl.pallas_call(
        functools.partial(_router_body, e=e, tm=TM),
        out_shape=(
            jax.ShapeDtypeStruct((n, LANES), jnp.float32),
            jax.ShapeDtypeStruct((e, LANES), jnp.float32),
            jax.ShapeDtypeStruct((n, LANES), jnp.int32),
            jax.ShapeDtypeStruct((nb_max, LANES), jnp.int32),
            jax.ShapeDtypeStruct((8, LANES), jnp.int32),
        ),
    )(x2d, wr_pad, br_pad)


def _sc_scatter2(rows, prob, idx, out_rows):
    b, d = rows.shape
    dp = prob.shape[1]
    info = plsc.get_sparse_core_info()
    nw = info.num_cores * info.num_subcores
    b_per_w = b // nw
    assert b_per_w * nw == b and b_per_w % 8 == 0
    mesh = plsc.VectorSubcoreMesh(core_axis_name="c", subcore_axis_name="s")

    @functools.partial(
        pl.kernel, mesh=mesh,
        out_type=(
            jax.ShapeDtypeStruct((out_rows, d), jnp.float32),
            jax.ShapeDtypeStruct((out_rows, dp), jnp.float32),
        ),
        scratch_types=[
            pltpu.VMEM((b_per_w,), jnp.int32),
            pltpu.VMEM((b_per_w, d), jnp.float32),
            pltpu.VMEM((b_per_w, dp), jnp.float32),
            pltpu.SemaphoreType.DMA,
            pltpu.SemaphoreType.DMA,
        ],
    )
    def k(rows_hbm, prob_hbm, idx_hbm, out_hbm, gate_hbm,
          idx_v, rows_v, prob_v, sem, semp):
        wid = lax.axis_index("s") * info.num_cores + lax.axis_index("c")
        base = wid * b_per_w
        pltpu.sync_copy(idx_hbm.at[pl.ds(base, b_per_w)], idx_v)
        pltpu.sync_copy(rows_hbm.at[pl.ds(base, b_per_w)], rows_v)
        pltpu.sync_copy(prob_hbm.at[pl.ds(base, b_per_w)], prob_v)
        cp1 = pltpu.async_copy(rows_v, out_hbm.at[idx_v], sem)
        cp2 = pltpu.async_copy(prob_v, gate_hbm.at[idx_v], semp)
        cp1.wait()
        cp2.wait()

    return k(rows, prob, idx)


def _sc_gather(table, idx):
    rows, d = table.shape
    b = idx.shape[0]
    info = plsc.get_sparse_core_info()
    nw = info.num_cores * info.num_subcores
    b_per_w = b // nw
    assert b_per_w * nw == b and b_per_w % 8 == 0
    mesh = plsc.VectorSubcoreMesh(core_axis_name="c", subcore_axis_name="s")

    @functools.partial(
        pl.kernel, mesh=mesh,
        out_type=jax.ShapeDtypeStruct((b, d), jnp.float32),
        scratch_types=[
            pltpu.VMEM((b_per_w,), jnp.int32),
            pltpu.VMEM((b_per_w, d), jnp.float32),
            pltpu.SemaphoreType.DMA,
        ],
    )
    def k(table_hbm, idx_hbm, out_hbm, idx_v, rows_v, sem):
        wid = lax.axis_index("s") * info.num_cores + lax.axis_index("c")
        base = wid * b_per_w
        pltpu.sync_copy(idx_hbm.at[pl.ds(base, b_per_w)], idx_v)
        pltpu.async_copy(table_hbm.at[idx_v], rows_v, sem).wait()
        pltpu.sync_copy(rows_v, out_hbm.at[pl.ds(base, b_per_w)])

    return k(table, idx)


def _ffn_body(be_ref, x_ref, w1_ref, b1_ref, w2_ref, b2_ref, gate_ref,
              out_ref, acc_ref, *, kt_total):
    kt = pl.program_id(1)
    x = x_ref[...].astype(jnp.bfloat16)
    w1 = w1_ref[0].astype(jnp.bfloat16)
    h = jnp.dot(x, w1, preferred_element_type=jnp.float32) + b1_ref[0]
    h = h * (1.0 / (1.0 + jnp.exp(-h)))
    w2 = w2_ref[0].astype(jnp.bfloat16)
    part = jnp.dot(h.astype(jnp.bfloat16), w2, preferred_element_type=jnp.float32)

    @pl.when(kt == 0)
    def _():
        acc_ref[...] = part

    @pl.when(kt > 0)
    def _():
        acc_ref[...] = acc_ref[...] + part

    @pl.when(kt == kt_total - 1)
    def _():
        out_ref[...] = (acc_ref[...] + b2_ref[0]) * gate_ref[:, 0:1]


def _run_ffn(x_sorted, w1, b1, w2, b2, gate_sorted, block_expert, num_blocks):
    n_pad, d = x_sorted.shape
    e, _, f = w1.shape
    kt_total = f // FT
    b1r = b1.reshape(e, 1, f)
    b2r = b2.reshape(e, 1, d)
    grid_spec = pltpu.PrefetchScalarGridSpec(
        num_scalar_prefetch=1,
        grid=(num_blocks, kt_total),
        in_specs=[
            pl.BlockSpec((TM, d), lambda b, kt, be: (b, 0)),
            pl.BlockSpec((1, d, FT), lambda b, kt, be: (be[b], 0, kt)),
            pl.BlockSpec((1, 1, FT), lambda b, kt, be: (be[b], 0, kt)),
            pl.BlockSpec((1, FT, d), lambda b, kt, be: (be[b], kt, 0)),
            pl.BlockSpec((1, 1, d), lambda b, kt, be: (be[b], 0, 0)),
            pl.BlockSpec((TM, LANES), lambda b, kt, be: (b, 0)),
        ],
        out_specs=pl.BlockSpec((TM, d), lambda b, kt, be: (b, 0)),
        scratch_shapes=[pltpu.VMEM((TM, d), jnp.float32)],
    )
    return pl.pallas_call(
        functools.partial(_ffn_body, kt_total=kt_total),
        grid_spec=grid_spec,
        out_shape=jax.ShapeDtypeStruct((n_pad, d), jnp.float32),
    )(block_expert, x_sorted, w1, b1r, w2, b2r, gate_sorted)


def kernel(x, Wr, br, W1, b1, W2, b2):
    bq, t, d = x.shape
    e = Wr.shape[1]
    n = bq * t
    n_pad = (((n + TM - 1) // TM) + e - 1) * TM
    nb_max = n_pad // TM
    nb_max_pad = ((nb_max + 7) // 8) * 8
    x2d = x.reshape(n, d)

    prob_m, aux_m, pos_m, be_m, nbl_m = _run_router(x2d, Wr, br, nb_max_pad)
    aux_loss = aux_m[0, 0]
    inv_pos = pos_m[:, 0]
    block_expert = be_m[:nb_max, 0]
    num_blocks = nbl_m[0, 0]

    x_sorted, gate_sorted = _sc_scatter2(x2d, prob_m, inv_pos, n_pad)
    y_sorted = _run_ffn(x_sorted, W1, b1, W2, b2, gate_sorted, block_expert,
                        num_blocks)
    out = _sc_gather(y_sorted, inv_pos)
    return out.reshape(bq, t, d), aux_loss


# speedup vs baseline: 2.8526x; 1.0217x over previous
import functools

import jax
import jax.numpy as jnp
from jax import lax
from jax.experimental import pallas as pl
from jax.experimental.pallas import tpu as pltpu
from jax.experimental.pallas import tpu_sc as plsc

TM = 320
FT = 2048
LANES = 128


def _router_body(x_ref, wr_ref, br_ref,
                 prob_ref, aux_ref, pos_ref, be_ref, nb_ref, *, e, tm):
    n, e_pad = prob_ref.shape
    x = x_ref[...]
    logits = jnp.dot(x, wr_ref[...], preferred_element_type=jnp.float32,
                     precision=lax.Precision.DEFAULT) + br_ref[...]
    m = jnp.max(logits, axis=1, keepdims=True)
    p = jnp.exp(logits - m)
    s = jnp.sum(p, axis=1, keepdims=True)
    lane = lax.broadcasted_iota(jnp.int32, (n, e_pad), 1)
    top1 = jnp.min(jnp.where(logits >= m, lane, e_pad), axis=1, keepdims=True)
    prob_ref[...] = jnp.broadcast_to(1.0 / s, (n, e_pad))

    probs = p / s
    onehot = jnp.where(lane == top1, 1.0, 0.0)
    importance = jnp.sum(probs, axis=0, keepdims=True) * (1.0 / n)
    load = jnp.sum(onehot, axis=0, keepdims=True) * (1.0 / n)
    aux = e * jnp.sum(importance * load)
    aux_ref[...] = jnp.broadcast_to(aux, aux_ref.shape)

    ch = 128
    nc = n // ch
    oh_bf = onehot.astype(jnp.bfloat16)
    rowc = lax.broadcasted_iota(jnp.int32, (nc, n), 0)
    colt = lax.broadcasted_iota(jnp.int32, (nc, n), 1)
    ci = jnp.where(rowc == colt // ch, 1.0, 0.0).astype(jnp.bfloat16)
    chunk_sums = jnp.dot(ci, oh_bf, preferred_element_type=jnp.float32)
    trilc = jnp.where(
        lax.broadcasted_iota(jnp.int32, (nc, nc), 0)
        > lax.broadcasted_iota(jnp.int32, (nc, nc), 1), 1.0, 0.0)
    pref_chunk = jnp.dot(trilc.astype(jnp.bfloat16),
                         chunk_sums.astype(jnp.bfloat16),
                         preferred_element_type=jnp.float32)
    tril128 = jnp.where(
        lax.broadcasted_iota(jnp.int32, (ch, ch), 0)
        > lax.broadcasted_iota(jnp.int32, (ch, ch), 1),
        1.0, 0.0).astype(jnp.bfloat16)
    prefix = jnp.concatenate(
        [jnp.dot(tril128, oh_bf[c * ch:(c + 1) * ch, :],
                 preferred_element_type=jnp.float32) + pref_chunk[c:c + 1, :]
         for c in range(nc)], axis=0)
    rank = jnp.sum(jnp.where(lane == top1, prefix, 0.0), axis=1, keepdims=True)

    counts = jnp.sum(onehot, axis=0, keepdims=True)
    nb = jnp.floor((counts + (tm - 1)) * (1.0 / tm))
    le_mask = jnp.where(
        lax.broadcasted_iota(jnp.int32, (e_pad, e_pad), 0)
        <= lax.broadcasted_iota(jnp.int32, (e_pad, e_pad), 1), 1.0, 0.0)
    cum_nb = jnp.dot(nb.astype(jnp.bfloat16), le_mask.astype(jnp.bfloat16),
                     preferred_element_type=jnp.float32)
    pos_start = (cum_nb - nb) * tm
    tok_start = jnp.sum(jnp.where(lane == top1, pos_start, 0.0),
                        axis=1, keepdims=True)
    pos = (tok_start + rank).astype(jnp.int32)
    pos_ref[...] = jnp.broadcast_to(pos, (n, e_pad))

    nbm = be_ref.shape[0]
    b_iota = lax.broadcasted_iota(jnp.int32, (nbm, e_pad), 0).astype(jnp.float32)
    in_e = jnp.where(lane[:nbm] < e, 1.0, 0.0)
    be = jnp.sum(jnp.where(b_iota >= jnp.broadcast_to(cum_nb, (nbm, e_pad)),
                           in_e, 0.0), axis=1, keepdims=True)
    be = jnp.minimum(be, float(e - 1)).astype(jnp.int32)
    be_ref[...] = jnp.broadcast_to(be, (nbm, e_pad))
    total = jnp.sum(nb * in_e[:1]).astype(jnp.int32)
    nb_ref[...] = jnp.broadcast_to(total, nb_ref.shape)


def _run_router(x2d, wr, br, nb_max):
    n, d = x2d.shape
    e = wr.shape[1]
    wr_pad = jnp.zeros((d, LANES), jnp.float32).at[:, :e].set(wr)
    br_pad = jnp.full((1, LANES), -1e30, jnp.float32).at[0, :e].set(br)
    return pl.pallas_call(
        functools.partial(_router_body, e=e, tm=TM),
        out_shape=(
            jax.ShapeDtypeStruct((n, LANES), jnp.float32),
            jax.ShapeDtypeStruct((e, LANES), jnp.float32),
            jax.ShapeDtypeStruct((n, LANES), jnp.int32),
            jax.ShapeDtypeStruct((nb_max, LANES), jnp.int32),
            jax.ShapeDtypeStruct((8, LANES), jnp.int32),
        ),
    )(x2d, wr_pad, br_pad)


def _sc_scatter2(rows, prob, idx, out_rows):
    b, d = rows.shape
    dp = prob.shape[1]
    info = plsc.get_sparse_core_info()
    nw = info.num_cores * info.num_subcores
    b_per_w = b // nw
    assert b_per_w * nw == b and b_per_w % 8 == 0
    mesh = plsc.VectorSubcoreMesh(core_axis_name="c", subcore_axis_name="s")

    @functools.partial(
        pl.kernel, mesh=mesh,
        out_type=(
            jax.ShapeDtypeStruct((out_rows, d), jnp.float32),
            jax.ShapeDtypeStruct((out_rows, dp), jnp.float32),
        ),
        scratch_types=[
            pltpu.VMEM((b_per_w,), jnp.int32),
            pltpu.VMEM((b_per_w, d), jnp.float32),
            pltpu.VMEM((b_per_w, dp), jnp.float32),
            pltpu.SemaphoreType.DMA,
            pltpu.SemaphoreType.DMA,
        ],
    )
    def k(rows_hbm, prob_hbm, idx_hbm, out_hbm, gate_hbm,
          idx_v, rows_v, prob_v, sem, semp):
        wid = lax.axis_index("s") * info.num_cores + lax.axis_index("c")
        base = wid * b_per_w
        pltpu.sync_copy(idx_hbm.at[pl.ds(base, b_per_w)], idx_v)
        pltpu.sync_copy(rows_hbm.at[pl.ds(base, b_per_w)], rows_v)
        pltpu.sync_copy(prob_hbm.at[pl.ds(base, b_per_w)], prob_v)
        cp1 = pltpu.async_copy(rows_v, out_hbm.at[idx_v], sem)
        cp2 = pltpu.async_copy(prob_v, gate_hbm.at[idx_v], semp)
        cp1.wait()
        cp2.wait()

    return k(rows, prob, idx)


def _sc_gather(table, idx):
    rows, d = table.shape
    b = idx.shape[0]
    info = plsc.get_sparse_core_info()
    nw = info.num_cores * info.num_subcores
    b_per_w = b // nw
    assert b_per_w * nw == b and b_per_w % 8 == 0
    mesh = plsc.VectorSubcoreMesh(core_axis_name="c", subcore_axis_name="s")

    @functools.partial(
        pl.kernel, mesh=mesh,
        out_type=jax.ShapeDtypeStruct((b, d), jnp.float32),
        scratch_types=[
            pltpu.VMEM((b_per_w,), jnp.int32),
            pltpu.VMEM((b_per_w, d), jnp.float32),
            pltpu.SemaphoreType.DMA,
        ],
    )
    def k(table_hbm, idx_hbm, out_hbm, idx_v, rows_v, sem):
        wid = lax.axis_index("s") * info.num_cores + lax.axis_index("c")
        base = wid * b_per_w
        pltpu.sync_copy(idx_hbm.at[pl.ds(base, b_per_w)], idx_v)
        pltpu.async_copy(table_hbm.at[idx_v], rows_v, sem).wait()
        pltpu.sync_copy(rows_v, out_hbm.at[pl.ds(base, b_per_w)])

    return k(table, idx)


def _ffn_body(be_ref, x_ref, w1_ref, b1_ref, w2_ref, b2_ref, gate_ref,
              out_ref, acc_ref, *, kt_total):
    kt = pl.program_id(1)
    x = x_ref[...].astype(jnp.bfloat16)
    w1 = w1_ref[0].astype(jnp.bfloat16)
    h = jnp.dot(x, w1, preferred_element_type=jnp.float32) + b1_ref[0]
    h = h * (1.0 / (1.0 + jnp.exp(-h)))
    w2 = w2_ref[0].astype(jnp.bfloat16)
    part = jnp.dot(h.astype(jnp.bfloat16), w2, preferred_element_type=jnp.float32)

    @pl.when(kt == 0)
    def _():
        acc_ref[...] = part

    @pl.when(kt > 0)
    def _():
        acc_ref[...] = acc_ref[...] + part

    @pl.when(kt == kt_total - 1)
    def _():
        out_ref[...] = (acc_ref[...] + b2_ref[0]) * gate_ref[:, 0:1]


def _run_ffn(x_sorted, w1, b1, w2, b2, gate_sorted, block_expert, num_blocks):
    n_pad, d = x_sorted.shape
    e, _, f = w1.shape
    kt_total = f // FT
    b1r = b1.reshape(e, 1, f)
    b2r = b2.reshape(e, 1, d)
    grid_spec = pltpu.PrefetchScalarGridSpec(
        num_scalar_prefetch=1,
        grid=(num_blocks, kt_total),
        in_specs=[
            pl.BlockSpec((TM, d), lambda b, kt, be: (b, 0)),
            pl.BlockSpec((1, d, FT), lambda b, kt, be: (be[b], 0, kt)),
            pl.BlockSpec((1, 1, FT), lambda b, kt, be: (be[b], 0, kt)),
            pl.BlockSpec((1, FT, d), lambda b, kt, be: (be[b], kt, 0)),
            pl.BlockSpec((1, 1, d), lambda b, kt, be: (be[b], 0, 0)),
            pl.BlockSpec((TM, LANES), lambda b, kt, be: (b, 0)),
        ],
        out_specs=pl.BlockSpec((TM, d), lambda b, kt, be: (b, 0)),
        scratch_shapes=[pltpu.VMEM((TM, d), jnp.float32)],
    )
    return pl.pallas_call(
        functools.partial(_ffn_body, kt_total=kt_total),
        grid_spec=grid_spec,
        out_shape=jax.ShapeDtypeStruct((n_pad, d), jnp.float32),
    )(block_expert, x_sorted, w1, b1r, w2, b2r, gate_sorted)


def kernel(x, Wr, br, W1, b1, W2, b2):
    bq, t, d = x.shape
    e = Wr.shape[1]
    n = bq * t
    n_pad = (((n + TM - 1) // TM) + e - 1) * TM
    nb_max = n_pad // TM
    nb_max_pad = ((nb_max + 7) // 8) * 8
    x2d = x.reshape(n, d)

    prob_m, aux_m, pos_m, be_m, nbl_m = _run_router(x2d, Wr, br, nb_max_pad)
    aux_loss = aux_m[0, 0]
    inv_pos = pos_m[:, 0]
    block_expert = be_m[:nb_max, 0]
    num_blocks = nbl_m[0, 0]

    x_sorted, gate_sorted = _sc_scatter2(x2d, prob_m, inv_pos, n_pad)
    y_sorted = _run_ffn(x_sorted, W1, b1, W2, b2, gate_sorted, block_expert,
                        num_blocks)
    out = _sc_gather(y_sorted, inv_pos)
    return out.reshape(bq, t, d), aux_loss
